# baseline (speedup 1.0000x reference)
import functools

import jax
import jax.numpy as jnp
from jax import lax
from jax.experimental import pallas as pl
from jax.experimental.pallas import tpu as pltpu
from jax.experimental.pallas import tpu_sc as plsc

N = 4096
P = 32
T = 64
ON_TIME = 10
DT = 0.1

NT = 16
CW = 128
NNZ = 838860
CH = -(-NNZ // (NT * CW))
NNZ_PAD = NT * CH * CW
FD = 8
SD = 16
ZW = 16384
NZ = (N * N) // (NT * ZW)


def _act(x):
    return 0.5 * (jnp.tanh(x) + 1.0)


def _densify_body(idx_hbm, vals_hbm, out_hbm, idx_v, vals_v, zero_v, sem,
                  sem2):
    sid = lax.axis_index("s")
    my_out = out_hbm

    pltpu.async_copy(idx_hbm.at[sid], idx_v, sem2)
    pltpu.async_copy(vals_hbm.at[sid], vals_v, sem2)

    def zbody(i, _):
        zero_v[pl.ds(i * 16, 16)] = jnp.zeros((16,), jnp.float32)
        return 0

    lax.fori_loop(0, ZW // 16, zbody, 0)

    base = sid * (N * N // NT)

    for b in range(FD):
        pltpu.async_copy(zero_v, my_out.at[pl.ds(base + b * ZW, ZW)], sem)

    def fsteady(j, _):
        pltpu.make_async_copy(zero_v, my_out.at[pl.ds(base, ZW)], sem).wait()
        pltpu.async_copy(zero_v, my_out.at[pl.ds(base + j * ZW, ZW)], sem)
        return 0

    lax.fori_loop(FD, NZ, fsteady, 0)
    for _b in range(FD):
        pltpu.make_async_copy(zero_v, my_out.at[pl.ds(base, ZW)], sem).wait()

    pltpu.make_async_copy(idx_hbm.at[sid], idx_v, sem2).wait()
    pltpu.make_async_copy(vals_hbm.at[sid], vals_v, sem2).wait()

    plsc.subcore_barrier()

    @plsc.parallel_loop(0, CH, unroll=8)
    def _scat(j):
        pltpu.async_copy(vals_v.at[j], my_out.at[idx_v.at[j]], sem)

    def sdrain(j, _):
        pltpu.make_async_copy(vals_v.at[0], my_out.at[idx_v.at[0]],
                              sem).wait()
        return 0

    lax.fori_loop(0, CH, sdrain, 0)


@functools.partial(
    pl.kernel,
    out_type=jax.ShapeDtypeStruct((N * N,), jnp.float32),
    mesh=plsc.VectorSubcoreMesh(core_axis_name="c", subcore_axis_name="s",
                                num_cores=1),
    scratch_types=[
        pltpu.VMEM((CH, CW), jnp.int32),
        pltpu.VMEM((CH, CW), jnp.float32),
        pltpu.VMEM((ZW,), jnp.float32),
        pltpu.SemaphoreType.DMA,
        pltpu.SemaphoreType.DMA,
    ],
)
def _densify(idx_hbm, vals_hbm, out_hbm, idx_v, vals_v, zero_v, sem, sem2):
    _densify_body(idx_hbm, vals_hbm, out_hbm, idx_v, vals_v, zero_v, sem,
                  sem2)


def _rnn_body(Jt_ref, pat_ref, m_ref, out_ref, x_ref):
    t = pl.program_id(0)

    @pl.when(t == 0)
    def _():
        x_ref[...] = jnp.zeros_like(x_ref)

    x = x_ref[...]
    rates = _act(x).astype(jnp.bfloat16)
    recur = jnp.dot(rates, Jt_ref[...], preferred_element_type=jnp.float32)
    inp = jnp.where(t < ON_TIME, pat_ref[...], 0.0)
    x_new = x + DT * (-x + recur + inp)
    x_ref[...] = x_new

    r_new = _act(x_new)
    out_ref[...] = jnp.dot(r_new, m_ref[...],
                           preferred_element_type=jnp.float32)[None]


def kernel(patterns, J_vals, w_out_vals, J_rows, J_cols, w_out_cols,
           N_time_steps):
    flat = J_cols.astype(jnp.int32) * N + J_rows.astype(jnp.int32)
    pad = NNZ_PAD - NNZ
    idx_p = jnp.concatenate([flat, jnp.broadcast_to(flat[:1], (pad,))])
    val_p = jnp.concatenate([J_vals, jnp.broadcast_to(J_vals[:1], (pad,))])
    idx_s = idx_p.reshape(NT, CH, CW)
    val_s = val_p.reshape(NT, CH, CW)

    Jt = _densify(idx_s, val_s).reshape(N, N).astype(jnp.bfloat16)

    hits = (jnp.arange(N, dtype=jnp.int32)[:, None] == w_out_cols[None, :])
    m = jnp.dot(hits.astype(jnp.float32), w_out_vals)
    m2 = m.reshape(N, 1)

    readout = pl.pallas_call(
        _rnn_body,
        grid=(T,),
        in_specs=[
            pl.BlockSpec((N, N), lambda t: (0, 0)),
            pl.BlockSpec((P, N), lambda t: (0, 0)),
            pl.BlockSpec((N, 1), lambda t: (0, 0)),
        ],
        out_specs=pl.BlockSpec((1, P, 1), lambda t: (t, 0, 0)),
        out_shape=jax.ShapeDtypeStruct((T, P, 1), jnp.float32),
        scratch_shapes=[
            pltpu.VMEM((P, N), jnp.float32),
        ],
    )(Jt, patterns.T, m2)

    return readout.reshape(T, P).T / N


# speedup vs baseline: 1.0035x; 1.0035x over previous
import functools

import jax
import jax.numpy as jnp
from jax import lax
from jax.experimental import pallas as pl
from jax.experimental.pallas import tpu as pltpu
from jax.experimental.pallas import tpu_sc as plsc

N = 4096
P = 32
T = 64
ON_TIME = 10
DT = 0.1

NT = 16
CW = 128
NNZ = 838860
CH = -(-NNZ // (NT * CW))
NNZ_PAD = NT * CH * CW
FD = 8
SD = 16
ZW = 16384
NZ = (N * N) // (NT * ZW)


def _act(x):
    return 0.5 * (jnp.tanh(x) + 1.0)


def _densify_body(idx_hbm, vals_hbm, out_hbm, idx_v, vals_v, zero_v, sem,
                  sem2):
    sid = lax.axis_index("s")
    my_out = out_hbm

    pltpu.async_copy(idx_hbm.at[sid], idx_v, sem2)
    pltpu.async_copy(vals_hbm.at[sid], vals_v, sem2)

    def zbody(i, _):
        zero_v[pl.ds(i * 16, 16)] = jnp.zeros((16,), jnp.float32)
        return 0

    lax.fori_loop(0, ZW // 16, zbody, 0)

    base = sid * (N * N // NT)

    for b in range(FD):
        pltpu.async_copy(zero_v, my_out.at[pl.ds(base + b * ZW, ZW)], sem)

    def fsteady(j, _):
        pltpu.make_async_copy(zero_v, my_out.at[pl.ds(base, ZW)], sem).wait()
        pltpu.async_copy(zero_v, my_out.at[pl.ds(base + j * ZW, ZW)], sem)
        return 0

    lax.fori_loop(FD, NZ, fsteady, 0)
    for _b in range(FD):
        pltpu.make_async_copy(zero_v, my_out.at[pl.ds(base, ZW)], sem).wait()

    pltpu.make_async_copy(idx_hbm.at[sid], idx_v, sem2).wait()
    pltpu.make_async_copy(vals_hbm.at[sid], vals_v, sem2).wait()

    plsc.subcore_barrier()

    for b in range(SD):
        pltpu.async_copy(vals_v.at[b], my_out.at[idx_v.at[b]], sem)

    def ssteady(j, _):
        pltpu.make_async_copy(vals_v.at[0], my_out.at[idx_v.at[0]],
                              sem).wait()
        pltpu.async_copy(vals_v.at[j], my_out.at[idx_v.at[j]], sem)
        return 0

    lax.fori_loop(SD, CH, ssteady, 0)
    for _b in range(SD):
        pltpu.make_async_copy(vals_v.at[0], my_out.at[idx_v.at[0]],
                              sem).wait()


@functools.partial(
    pl.kernel,
    out_type=jax.ShapeDtypeStruct((N * N,), jnp.float32),
    mesh=plsc.VectorSubcoreMesh(core_axis_name="c", subcore_axis_name="s",
                                num_cores=1),
    scratch_types=[
        pltpu.VMEM((CH, CW), jnp.int32),
        pltpu.VMEM((CH, CW), jnp.float32),
        pltpu.VMEM((ZW,), jnp.float32),
        pltpu.SemaphoreType.DMA,
        pltpu.SemaphoreType.DMA,
    ],
)
def _densify(idx_hbm, vals_hbm, out_hbm, idx_v, vals_v, zero_v, sem, sem2):
    _densify_body(idx_hbm, vals_hbm, out_hbm, idx_v, vals_v, zero_v, sem,
                  sem2)


CR = 256
NCH = N // CR


def _rnn_body(Jf_hbm, pat_ref, m_ref, out_ref, x_ref, jt_ref, fbuf_ref, sem):
    t = pl.program_id(0)

    @pl.when(t == 0)
    def _():
        x_ref[...] = jnp.zeros_like(x_ref)

        def cbody(k, _):
            slot = lax.rem(k, 2)
            nxt = lax.rem(k + 1, 2)

            @pl.when(k == 0)
            def _():
                pltpu.make_async_copy(Jf_hbm.at[pl.ds(0, CR), :],
                                      fbuf_ref.at[0], sem.at[0]).start()

            @pl.when(k + 1 < NCH)
            def _():
                pltpu.make_async_copy(Jf_hbm.at[pl.ds((k + 1) * CR, CR), :],
                                      fbuf_ref.at[nxt], sem.at[nxt]).start()

            pltpu.make_async_copy(Jf_hbm.at[pl.ds(k * CR, CR), :],
                                  fbuf_ref.at[slot], sem.at[slot]).wait()
            jt_ref[pl.ds(k * CR, CR), :] = fbuf_ref[slot].astype(jnp.bfloat16)
            return 0

        lax.fori_loop(0, NCH, cbody, 0)

    x = x_ref[...]
    rates = _act(x).astype(jnp.bfloat16)
    recur = jnp.dot(rates, jt_ref[...], preferred_element_type=jnp.float32)
    inp = jnp.where(t < ON_TIME, pat_ref[...], 0.0)
    x_new = x + DT * (-x + recur + inp)
    x_ref[...] = x_new

    r_new = _act(x_new)
    out_ref[...] = jnp.dot(r_new, m_ref[...],
                           preferred_element_type=jnp.float32)[None]


def kernel(patterns, J_vals, w_out_vals, J_rows, J_cols, w_out_cols,
           N_time_steps):
    flat = J_cols.astype(jnp.int32) * N + J_rows.astype(jnp.int32)
    pad = NNZ_PAD - NNZ
    idx_p = jnp.concatenate([flat, jnp.broadcast_to(flat[:1], (pad,))])
    val_p = jnp.concatenate([J_vals, jnp.broadcast_to(J_vals[:1], (pad,))])
    idx_s = idx_p.reshape(NT, CH, CW)
    val_s = val_p.reshape(NT, CH, CW)

    Jt = _densify(idx_s, val_s).reshape(N, N)

    hits = (jnp.arange(N, dtype=jnp.int32)[:, None] == w_out_cols[None, :])
    m = jnp.dot(hits.astype(jnp.float32), w_out_vals)
    m2 = m.reshape(N, 1)

    readout = pl.pallas_call(
        _rnn_body,
        grid=(T,),
        in_specs=[
            pl.BlockSpec(memory_space=pl.ANY),
            pl.BlockSpec((P, N), lambda t: (0, 0)),
            pl.BlockSpec((N, 1), lambda t: (0, 0)),
        ],
        out_specs=pl.BlockSpec((1, P, 1), lambda t: (t, 0, 0)),
        out_shape=jax.ShapeDtypeStruct((T, P, 1), jnp.float32),
        scratch_shapes=[
            pltpu.VMEM((P, N), jnp.float32),
            pltpu.VMEM((N, N), jnp.bfloat16),
            pltpu.VMEM((2, CR, N), jnp.float32),
            pltpu.SemaphoreType.DMA((2,)),
        ],
    )(Jt, patterns.T, m2)

    return readout.reshape(T, P).T / N
